# Initial kernel scaffold; baseline (speedup 1.0000x reference)
import jax
import jax.numpy as jnp
from jax.experimental import pallas as pl


def kernel(pred_labels, true_labels):
    raise NotImplementedError("write your pallas kernel here")


# speedup vs baseline: 25.8450x; 25.8450x over previous
import numpy as np
import jax
import jax.numpy as jnp
from jax.experimental import pallas as pl

_EPS = float(np.finfo(np.float32).eps)
_R = 2


def _shift(x, dr, dc):
    if dr:
        x = jnp.roll(x, -dr, axis=0)
    if dc:
        x = jnp.roll(x, -dc, axis=1)
    return x


def _nms_cell(pred_ref, lab_ref, out_ref):
    c = pl.program_id(1)
    pred = pred_ref[0, 0]
    lab = lab_ref[0]
    H, W = pred.shape

    m = (lab == c).astype(jnp.float32)

    m_u2 = _shift(m, -2, 0)
    m_u1 = _shift(m, -1, 0)
    m_d1 = _shift(m, 1, 0)
    m_d2 = _shift(m, 2, 0)
    a_s2 = (m_u2 + m_d2 + 4.0 * (m_u1 + m_d1) + 6.0 * m) * (1.0 / 16.0)
    a_d2 = (m_u2 + m_d2 - 2.0 * m) * 0.25
    a_sd = (-m_u2 - 2.0 * m_u1 + 2.0 * m_d1 + m_d2) * 0.125

    grad_xx = (_shift(a_s2, 0, -2) + _shift(a_s2, 0, 2) - 2.0 * a_s2) * 0.25
    grad_yy = (_shift(a_d2, 0, -2) + _shift(a_d2, 0, 2)
               + 4.0 * (_shift(a_d2, 0, -1) + _shift(a_d2, 0, 1))
               + 6.0 * a_d2) * (1.0 / 16.0)
    grad_xy = (-_shift(a_sd, 0, -2) - 2.0 * _shift(a_sd, 0, -1)
               + 2.0 * _shift(a_sd, 0, 1) + _shift(a_sd, 0, 2)) * 0.125

    theta = jnp.arctan2(
        grad_yy * jnp.sign(-grad_xy + _EPS) / (grad_xx + _EPS), 1.0)
    thresh = jnp.fmod(jnp.round(theta * (5.0 / np.pi)) + 5.0, 5.0)
    d = jnp.fmod(thresh, 4.0)

    ep = jnp.exp(pred)
    denom_h = _shift(ep, 0, -2) + _shift(ep, 0, -1) + ep + _shift(ep, 0, 1)
    denom_v = _shift(ep, -2, 0) + _shift(ep, -1, 0) + ep + _shift(ep, 1, 0)
    denom_d = (_shift(ep, -2, 1) + _shift(ep, -1, 0)
               + _shift(ep, 0, -1) + _shift(ep, 1, -2))

    val = jnp.where(d == 0.0, ep / denom_h,
                    jnp.where(d == 2.0, ep / denom_v, ep / denom_d))

    ri = jax.lax.broadcasted_iota(jnp.int32, (H, W), 0)
    ci = jax.lax.broadcasted_iota(jnp.int32, (H, W), 1)
    interior = (ri >= _R) & (ri < H - _R) & (ci >= _R) & (ci < W - _R)
    out_ref[0, 0] = jnp.sum(jnp.where(interior, val, 0.0)).reshape(1, 1)


def kernel(pred_labels, true_labels):
    B, C, H, W = pred_labels.shape
    labels = true_labels.astype(jnp.int32)
    partials = pl.pallas_call(
        _nms_cell,
        grid=(B, C),
        in_specs=[
            pl.BlockSpec((1, 1, H, W), lambda b, c: (b, c, 0, 0)),
            pl.BlockSpec((1, H, W), lambda b, c: (b, 0, 0)),
        ],
        out_specs=pl.BlockSpec((1, 1, 1, 1), lambda b, c: (b, c, 0, 0)),
        out_shape=jax.ShapeDtypeStruct((B, C, 1, 1), jnp.float32),
    )(pred_labels, labels)
    return jnp.sum(partials)


# speedup vs baseline: 33.0124x; 1.2773x over previous
import numpy as np
import jax
import jax.numpy as jnp
from jax.experimental import pallas as pl

_EPS = float(np.finfo(np.float32).eps)
_R = 2


def _shift(x, dr, dc):
    if dr:
        x = jnp.roll(x, -dr, axis=0)
    if dc:
        x = jnp.roll(x, -dc, axis=1)
    return x


def _nms_cell(pred_ref, lab_ref, out_ref):
    c = pl.program_id(1)
    pred = pred_ref[0, 0]
    lab = lab_ref[0]
    H, W = pred.shape

    m = (lab == c).astype(jnp.float32)

    m_u2 = _shift(m, -2, 0)
    m_u1 = _shift(m, -1, 0)
    m_d1 = _shift(m, 1, 0)
    m_d2 = _shift(m, 2, 0)
    a_s2 = (m_u2 + m_d2 + 4.0 * (m_u1 + m_d1) + 6.0 * m) * (1.0 / 16.0)
    a_d2 = (m_u2 + m_d2 - 2.0 * m) * 0.25
    a_sd = (-m_u2 - 2.0 * m_u1 + 2.0 * m_d1 + m_d2) * 0.125

    grad_xx = (_shift(a_s2, 0, -2) + _shift(a_s2, 0, 2) - 2.0 * a_s2) * 0.25
    grad_yy = (_shift(a_d2, 0, -2) + _shift(a_d2, 0, 2)
               + 4.0 * (_shift(a_d2, 0, -1) + _shift(a_d2, 0, 1))
               + 6.0 * a_d2) * (1.0 / 16.0)
    grad_xy = (-_shift(a_sd, 0, -2) - 2.0 * _shift(a_sd, 0, -1)
               + 2.0 * _shift(a_sd, 0, 1) + _shift(a_sd, 0, 2)) * 0.125

    s = jnp.where(grad_xy < _EPS, 1.0, -1.0)
    z = grad_yy * s / (grad_xx + _EPS)
    t1 = 0.3249196962329063
    t3 = 1.3763819204711735
    is_h = (z >= -t3) & (z < t1)
    is_v = z >= t3

    ep = jnp.exp(pred)
    denom_h = _shift(ep, 0, -2) + _shift(ep, 0, -1) + ep + _shift(ep, 0, 1)
    denom_v = _shift(ep, -2, 0) + _shift(ep, -1, 0) + ep + _shift(ep, 1, 0)
    denom_d = (_shift(ep, -2, 1) + _shift(ep, -1, 0)
               + _shift(ep, 0, -1) + _shift(ep, 1, -2))

    denom = jnp.where(is_h, denom_h, jnp.where(is_v, denom_v, denom_d))
    val = ep / denom

    ri = jax.lax.broadcasted_iota(jnp.int32, (H, W), 0)
    ci = jax.lax.broadcasted_iota(jnp.int32, (H, W), 1)
    interior = (ri >= _R) & (ri < H - _R) & (ci >= _R) & (ci < W - _R)
    out_ref[0, 0] = jnp.sum(jnp.where(interior, val, 0.0)).reshape(1, 1)


def kernel(pred_labels, true_labels):
    B, C, H, W = pred_labels.shape
    labels = true_labels.astype(jnp.int32)
    partials = pl.pallas_call(
        _nms_cell,
        grid=(B, C),
        in_specs=[
            pl.BlockSpec((1, 1, H, W), lambda b, c: (b, c, 0, 0)),
            pl.BlockSpec((1, H, W), lambda b, c: (b, 0, 0)),
        ],
        out_specs=pl.BlockSpec((1, 1, 1, 1), lambda b, c: (b, c, 0, 0)),
        out_shape=jax.ShapeDtypeStruct((B, C, 1, 1), jnp.float32),
    )(pred_labels, labels)
    return jnp.sum(partials)


# speedup vs baseline: 44.8581x; 1.3588x over previous
import numpy as np
import jax
import jax.numpy as jnp
from jax.experimental import pallas as pl

_EPS = float(np.finfo(np.float32).eps)
_R = 2


def _shift(x, dr, dc):
    if dr:
        x = jnp.roll(x, -dr, axis=0)
    if dc:
        x = jnp.roll(x, -dc, axis=1)
    return x


def _nms_cell(pred_ref, lab_ref, out_ref):
    c = pl.program_id(1)
    pred = pred_ref[0, 0]
    lab = lab_ref[0]
    H, W = pred.shape

    m = (lab == c).astype(jnp.bfloat16)

    m_u2 = _shift(m, -2, 0)
    m_u1 = _shift(m, -1, 0)
    m_d1 = _shift(m, 1, 0)
    m_d2 = _shift(m, 2, 0)
    a_s2 = (m_u2 + m_d2 + 4.0 * (m_u1 + m_d1) + 6.0 * m) * (1.0 / 16.0)
    a_d2 = (m_u2 + m_d2 - 2.0 * m) * 0.25
    a_sd = (-m_u2 - 2.0 * m_u1 + 2.0 * m_d1 + m_d2) * 0.125

    grad_xx = (_shift(a_s2, 0, -2) + _shift(a_s2, 0, 2) - 2.0 * a_s2) * 0.25
    grad_yy = (_shift(a_d2, 0, -2) + _shift(a_d2, 0, 2)
               + 4.0 * (_shift(a_d2, 0, -1) + _shift(a_d2, 0, 1))
               + 6.0 * a_d2) * (1.0 / 16.0)
    grad_xy = (-_shift(a_sd, 0, -2) - 2.0 * _shift(a_sd, 0, -1)
               + 2.0 * _shift(a_sd, 0, 1) + _shift(a_sd, 0, 2)) * 0.125

    grad_xx = grad_xx.astype(jnp.float32)
    grad_yy = grad_yy.astype(jnp.float32)
    grad_xy = grad_xy.astype(jnp.float32)
    s = jnp.where(grad_xy < _EPS, 1.0, -1.0)
    z = grad_yy * s / (grad_xx + _EPS)
    t1 = 0.3249196962329063
    t3 = 1.3763819204711735
    is_h = (z >= -t3) & (z < t1)
    is_v = z >= t3

    ep = jnp.exp(pred)
    denom_h = _shift(ep, 0, -2) + _shift(ep, 0, -1) + ep + _shift(ep, 0, 1)
    denom_v = _shift(ep, -2, 0) + _shift(ep, -1, 0) + ep + _shift(ep, 1, 0)
    denom_d = (_shift(ep, -2, 1) + _shift(ep, -1, 0)
               + _shift(ep, 0, -1) + _shift(ep, 1, -2))

    denom = jnp.where(is_h, denom_h, jnp.where(is_v, denom_v, denom_d))
    val = ep / denom

    ri = jax.lax.broadcasted_iota(jnp.int32, (H, W), 0)
    ci = jax.lax.broadcasted_iota(jnp.int32, (H, W), 1)
    interior = (ri >= _R) & (ri < H - _R) & (ci >= _R) & (ci < W - _R)
    out_ref[0, 0] = jnp.sum(jnp.where(interior, val, 0.0)).reshape(1, 1)


def kernel(pred_labels, true_labels):
    B, C, H, W = pred_labels.shape
    labels = true_labels.astype(jnp.int32)
    partials = pl.pallas_call(
        _nms_cell,
        grid=(B, C),
        in_specs=[
            pl.BlockSpec((1, 1, H, W), lambda b, c: (b, c, 0, 0)),
            pl.BlockSpec((1, H, W), lambda b, c: (b, 0, 0)),
        ],
        out_specs=pl.BlockSpec((1, 1, 1, 1), lambda b, c: (b, c, 0, 0)),
        out_shape=jax.ShapeDtypeStruct((B, C, 1, 1), jnp.float32),
    )(pred_labels, labels)
    return jnp.sum(partials)
